# baseline (speedup 1.0000x reference)
import jax
import jax.numpy as jnp
from jax import lax
from jax.experimental import pallas as pl
from jax.experimental.pallas import tpu as pltpu
from jax.experimental.pallas import tpu_sc as plsc

N = 10000
E = 320000
CIN = 128
H = 128
OUTC = 64
K = 9
PAD = (K - 1) // 2

BN = 1000
GRID = N // BN

NC = 2
NS = 16
NW = NC * NS
EPW = E // NW
CH = 80
NCHUNK = EPW // CH
RPT = 632
NP = RPT * NS

_sc_mesh = plsc.VectorSubcoreMesh(core_axis_name="c", subcore_axis_name="s")


NB = 3
PH = 5
PHN = NCHUNK // PH
PHG = PHN // NB
PHTAIL = PHN - PHG * NB

_scatter_scratch = (
    [pltpu.VMEM((PHN, CH), jnp.int32) for _ in range(4)]
    + [pltpu.VMEM_SHARED((NP, H), jnp.float32)]
    + [pltpu.VMEM((CH, H), jnp.float32) for _ in range(NB)]
    + [pltpu.SemaphoreType.DMA for _ in range(NB + 1)]
)


@pl.kernel(
    mesh=_sc_mesh,
    out_type=jax.ShapeDtypeStruct((NC, NP, H), jnp.float32),
    scratch_types=_scatter_scratch,
)
def _sc_scatter(h_hbm, src_hbm, dst_hbm, z128_hbm, acc_out,
                sidx0, didx0, sidx1, didx1, acc_sh, *rest):
    bufs = rest[:NB]
    sems = rest[NB:2 * NB]
    sem_i = rest[2 * NB]
    c = lax.axis_index("c")
    s = lax.axis_index("s")
    wid = c * NS + s
    r0 = s * RPT
    pltpu.sync_copy(src_hbm.at[wid, 0], sidx0)
    pltpu.sync_copy(dst_hbm.at[wid, 0], didx0)
    pltpu.sync_copy(z128_hbm, acc_sh.at[pl.ds(r0, RPT)])
    plsc.subcore_barrier()

    def ring_phase(sidx, didx):
        for b in range(NB):
            pltpu.async_copy(h_hbm.at[sidx.at[b]], bufs[b], sems[b])

        def body(g, carry):
            for b in range(NB):
                jj = g * NB + b
                pltpu.make_async_copy(h_hbm.at[sidx.at[jj]], bufs[b],
                                      sems[b]).wait()
                pltpu.sync_copy(bufs[b], acc_sh.at[didx.at[jj]], add=True)

                @pl.when(jj + NB < PHN)
                def _():
                    pltpu.async_copy(h_hbm.at[sidx.at[jj + NB]], bufs[b],
                                     sems[b])
            return carry

        lax.fori_loop(0, PHG, body, 0)
        for b in range(PHTAIL):
            jj = PHG * NB + b
            pltpu.make_async_copy(h_hbm.at[sidx.at[jj]], bufs[b],
                                  sems[b]).wait()
            pltpu.sync_copy(bufs[b], acc_sh.at[didx.at[jj]], add=True)

    idxs = [(sidx0, didx0), (sidx1, didx1)]
    for p in range(PH):
        cur = idxs[p % 2]
        nxt = idxs[(p + 1) % 2]
        if p + 1 < PH:
            pltpu.async_copy(src_hbm.at[wid, p + 1], nxt[0], sem_i)
            pltpu.async_copy(dst_hbm.at[wid, p + 1], nxt[1], sem_i)
        ring_phase(*cur)
        if p + 1 < PH:
            pltpu.make_async_copy(src_hbm.at[wid, p + 1], nxt[0],
                                  sem_i).wait()
            pltpu.make_async_copy(dst_hbm.at[wid, p + 1], nxt[1],
                                  sem_i).wait()

    plsc.subcore_barrier()
    pltpu.sync_copy(acc_sh.at[pl.ds(r0, RPT)], acc_out.at[c, pl.ds(r0, RPT)])


NPC_R = 80
CHC = 80
PHC = 5
PHNC = EPW // (CHC * PHC)


@pl.kernel(
    mesh=_sc_mesh,
    out_type=jax.ShapeDtypeStruct((NC, NPC_R, H), jnp.float32),
    scratch_types=(
        [pltpu.VMEM((PHNC, CHC), jnp.int32) for _ in range(2)]
        + [pltpu.VMEM((NPC_R, H), jnp.float32),
           pltpu.VMEM((NPC_R,), jnp.int32),
           pltpu.VMEM_SHARED((NPC_R, H), jnp.float32)]
        + [pltpu.SemaphoreType.DMA]
    ),
    compiler_params=pltpu.CompilerParams(needs_layout_passes=False),
)
def _sc_count(dst_hbm, z128_hbm, cnt_out, didx0, didx1, cnt2, ridx, cnt_sh,
              sem_i):
    c = lax.axis_index("c")
    s = lax.axis_index("s")
    wid = c * NS + s
    pltpu.sync_copy(dst_hbm.at[wid, 0], didx0)
    pltpu.sync_copy(z128_hbm.at[pl.ds(0, NPC_R)], cnt2)
    for i in range(NPC_R // 16):
        ridx[pl.ds(i * 16, 16)] = lax.iota(jnp.int32, 16) + i * 16

    @pl.when(s == 0)
    def _():
        pltpu.sync_copy(z128_hbm.at[pl.ds(0, NPC_R)], cnt_sh)
    plsc.subcore_barrier()

    ones16 = jnp.full((16,), 1.0, jnp.float32)

    def count_phase(didx):
        def body(jj, carry):
            for t in range(CHC // 16):
                d = didx[jj, pl.ds(t * 16, 16)]
                row = lax.shift_right_logical(d, 7)
                col = jnp.bitwise_and(d, 127)
                plsc.addupdate_scatter(cnt2, [row, col], ones16)
            return carry

        lax.fori_loop(0, PHNC, body, 0)

    idxs = [didx0, didx1]
    for p in range(PHC):
        if p + 1 < PHC:
            pltpu.async_copy(dst_hbm.at[wid, p + 1], idxs[(p + 1) % 2],
                             sem_i)
        count_phase(idxs[p % 2])
        if p + 1 < PHC:
            pltpu.make_async_copy(dst_hbm.at[wid, p + 1], idxs[(p + 1) % 2],
                                  sem_i).wait()

    pltpu.sync_copy(cnt2, cnt_sh.at[ridx], add=True)
    plsc.subcore_barrier()

    @pl.when(s < 10)
    def _():
        pltpu.sync_copy(cnt_sh.at[pl.ds(s * 8, 8)],
                        cnt_out.at[c, pl.ds(s * 8, 8)])


def _conv_body(h_ref, w_ref, b_ref, o_ref):
    i = pl.program_id(0)

    def emit(delta, w0):
        hp = h_ref[pl.ds(w0, BN + 16), :]
        ext = jnp.pad(hp, ((4, 4), (0, 0)))
        xs = [lax.slice(ext, (delta + k, 0), (delta + k + BN, CIN))
              for k in range(K)]
        xcat = jnp.concatenate(xs, axis=1)
        acc = jnp.dot(xcat, w_ref[...],
                      preferred_element_type=jnp.float32) + b_ref[...]
        o_ref[...] = jnp.maximum(acc, 0.0)

    @pl.when(i == 0)
    def _():
        emit(0, 0)

    @pl.when(jnp.logical_and(i > 0, i < GRID - 1))
    def _():
        emit(8, pl.multiple_of(i * BN - 8, 8))

    @pl.when(i == GRID - 1)
    def _():
        emit(16, N - BN - 16)


def _tc_conv(h, w, b):
    return pl.pallas_call(
        _conv_body,
        grid=(GRID,),
        in_specs=[
            pl.BlockSpec((N, CIN), lambda i: (0, 0)),
            pl.BlockSpec((K * CIN, H), lambda i: (0, 0)),
            pl.BlockSpec((1, H), lambda i: (0, 0)),
        ],
        out_specs=pl.BlockSpec((BN, H), lambda i: (i, 0)),
        out_shape=jax.ShapeDtypeStruct((N, H), jnp.float32),
    )(h, w, b)


def _sage_block(acc_ref, cnt_ref, hc, wlr, bl, gs, be):
    a = acc_ref[...]
    agg = a[0] + a[1]
    c = cnt_ref[...]
    cnt = c[:, 0:1] + c[:, 1:2]
    mean = agg * (1.0 / jnp.maximum(cnt, 1.0))
    z = jnp.dot(jnp.concatenate([mean, hc[...]], axis=1), wlr[...],
                preferred_element_type=jnp.float32) + bl[...]
    return jnp.maximum(gs[...] * z + be[...], 0.0)


def _fused_body(acc_ref, cnt_ref, hc_ref, wlr, bl, gs, be, wc_ref, tb_ref,
                o_ref):
    i = pl.program_id(0)

    def emit(delta, w0):
        a = acc_ref[:, pl.ds(w0, BN + 16), :]
        agg = a[0] + a[1]
        cpair = cnt_ref[pl.ds(w0, BN + 16), :]
        cnt = cpair[:, 0:1] + cpair[:, 1:2]
        mean = agg * (1.0 / jnp.maximum(cnt, 1.0))
        hcv = hc_ref[pl.ds(w0, BN + 16), :]
        z = jnp.dot(jnp.concatenate([mean, hcv], axis=1), wlr[...],
                    preferred_element_type=jnp.float32) + bl[...]
        h2 = jnp.maximum(gs[...] * z + be[...], 0.0)
        ext = jnp.pad(h2, ((4, 4), (0, 0)))
        xs = [lax.slice(ext, (delta + k, 0), (delta + k + BN, CIN))
              for k in range(K)]
        xcat = jnp.concatenate(xs, axis=1)
        acv = jnp.dot(xcat, wc_ref[...],
                      preferred_element_type=jnp.float32) + tb_ref[...]
        o_ref[...] = jnp.maximum(acv, 0.0)

    @pl.when(i == 0)
    def _():
        emit(0, 0)

    @pl.when(jnp.logical_and(i > 0, i < GRID - 1))
    def _():
        emit(8, pl.multiple_of(i * BN - 8, 8))

    @pl.when(i == GRID - 1)
    def _():
        emit(16, N - BN - 16)


def _tc_fused(accp, cntp, hc, wlr, bl, gs, be, wc, tb):
    return pl.pallas_call(
        _fused_body,
        grid=(GRID,),
        in_specs=[
            pl.BlockSpec((NC, NP, H), lambda i: (0, 0, 0)),
            pl.BlockSpec((N, 2), lambda i: (0, 0)),
            pl.BlockSpec((N, H), lambda i: (0, 0)),
            pl.BlockSpec((2 * H, H), lambda i: (0, 0)),
            pl.BlockSpec((1, H), lambda i: (0, 0)),
            pl.BlockSpec((1, H), lambda i: (0, 0)),
            pl.BlockSpec((1, H), lambda i: (0, 0)),
            pl.BlockSpec((K * CIN, H), lambda i: (0, 0)),
            pl.BlockSpec((1, H), lambda i: (0, 0)),
        ],
        out_specs=pl.BlockSpec((BN, H), lambda i: (i, 0)),
        out_shape=jax.ShapeDtypeStruct((N, H), jnp.float32),
    )(accp, cntp, hc, wlr, bl, gs, be, wc, tb)


def _final_body(acc_ref, cnt_ref, hc, wlr, bl, gs, be, cw_ref, cb_ref,
                o_ref):
    h2 = _sage_block(acc_ref, cnt_ref, hc, wlr, bl, gs, be)
    logits = jnp.dot(h2, cw_ref[...], preferred_element_type=jnp.float32) \
        + cb_ref[...]
    m = jnp.max(logits, axis=1, keepdims=True)
    lse = jnp.log(jnp.sum(jnp.exp(logits - m), axis=1, keepdims=True)) + m
    o_ref[...] = logits - lse


def _sage_specs():
    return [
        pl.BlockSpec((NC, BN, H), lambda i: (0, i, 0)),
        pl.BlockSpec((BN, 2), lambda i: (i, 0)),
        pl.BlockSpec((BN, H), lambda i: (i, 0)),
        pl.BlockSpec((2 * H, H), lambda i: (0, 0)),
        pl.BlockSpec((1, H), lambda i: (0, 0)),
        pl.BlockSpec((1, H), lambda i: (0, 0)),
        pl.BlockSpec((1, H), lambda i: (0, 0)),
    ]


def _tc_final(accp, cntp, hc, wlr, bl, gs, be, cw, cb):
    return pl.pallas_call(
        _final_body,
        grid=(GRID,),
        in_specs=_sage_specs() + [
            pl.BlockSpec((H, OUTC), lambda i: (0, 0)),
            pl.BlockSpec((1, OUTC), lambda i: (0, 0)),
        ],
        out_specs=pl.BlockSpec((BN, OUTC), lambda i: (i, 0)),
        out_shape=jax.ShapeDtypeStruct((N, OUTC), jnp.float32),
    )(accp, cntp, hc, wlr, bl, gs, be, cw, cb)


def kernel(x, edge_index, tw0, tb0, swl0, sbl0, swr0, g0, be0,
           tw1, tb1, swl1, sbl1, swr1, g1, be1,
           tw2, tb2, swl2, sbl2, swr2, g2, be2, cw, cb):
    f32 = jnp.float32
    src3 = edge_index[0].reshape(NW, PH, PHN, CH)
    dst3 = edge_index[1].reshape(NW, PH, PHN, CH)
    dst3c = edge_index[1].reshape(NW, PHC, PHNC, CHC)
    z128 = jnp.zeros((RPT, H), f32)
    gscale = 1.0 / jnp.sqrt(jnp.float32(1.0 + 1e-5))

    layers = []
    for tw, tb, wl, bl, wr, g, be in (
            (tw0, tb0, swl0, sbl0, swr0, g0, be0),
            (tw1, tb1, swl1, sbl1, swr1, g1, be1),
            (tw2, tb2, swl2, sbl2, swr2, g2, be2)):
        layers.append((
            jnp.transpose(tw, (2, 1, 0)).reshape(K * CIN, H),
            tb.reshape(1, H),
            jnp.concatenate([wl.T, wr.T], axis=0), bl.reshape(1, H),
            (g * gscale).reshape(1, H), be.reshape(1, H),
        ))
    cwT = cw.T
    cbr = cb.reshape(1, OUTC)

    h = x
    cnt_raw = _sc_count(dst3c, z128)
    cntp = jnp.transpose(cnt_raw.reshape(NC, NPC_R * H)[:, :N])
    hc = _tc_conv(x, layers[0][0], layers[0][1])
    for li in range(3):
        wk, tb, wlr, bl, gs, be = layers[li]
        accp = _sc_scatter(hc, src3, dst3, z128)
        if li < 2:
            nwk, ntb = layers[li + 1][0], layers[li + 1][1]
            hc = _tc_fused(accp, cntp, hc, wlr, bl, gs, be, nwk, ntb)
        else:
            out = _tc_final(accp, cntp, hc, wlr, bl, gs, be, cwT, cbr)
    return out
